# baseline (speedup 1.0000x reference)
import functools
import math
from collections import Counter
from itertools import combinations_with_replacement as cwr

import numpy as np
import jax
import jax.numpy as jnp
from jax import lax
from jax.experimental import pallas as pl
from jax.experimental.pallas import tpu as pltpu
from jax.experimental.pallas import tpu_sc as plsc

N = 8192
F = 64
D = 4
L = 4
S = 10
N1, N2, N3 = 4, 8, 12
NB = N1 + N2 + N3

N_SC = 2048
N_TC = N - N_SC
BN_TC = 256

NW = 32
NPW = N_SC // NW
TILE = 64
NT = NPW // TILE

LANES = 16
FC = F // LANES


def _build_monos_and_cml():
    rng = np.random.RandomState(42)
    P = (rng.randn(NB, NB) / np.sqrt(NB)).astype(np.float32)
    U1 = rng.randn(D, L, N1).astype(np.float32)
    U2 = rng.randn(D, D, L, N2).astype(np.float32)
    U2 = 0.5 * (U2 + U2.transpose(1, 0, 2, 3))
    U3 = rng.randn(D, D, D, L, N3).astype(np.float32)
    perms = [(0, 1, 2), (0, 2, 1), (1, 0, 2), (1, 2, 0), (2, 0, 1), (2, 1, 0)]
    U3 = sum(U3.transpose(p + (3, 4)) for p in perms) / 6.0

    K1 = np.einsum('ilb,ab->ila', U1, P[:, :N1])
    K2 = np.einsum('ijlb,ab->ijla', U2, P[:, N1:N1 + N2])
    K3 = np.einsum('ijklb,ab->ijkla', U3, P[:, N1 + N2:])

    monos = []
    rows = []
    for i in range(D):
        monos.append((i,))
        rows.append(K1[i])
    for (i, j) in cwr(range(D), 2):
        mult = 1 if i == j else 2
        monos.append((i, j))
        rows.append(mult * K2[i, j])
    for t in cwr(range(D), 3):
        c = Counter(t)
        mult = math.factorial(3)
        for v in c.values():
            mult //= math.factorial(v)
        monos.append(t)
        rows.append(mult * K3[t[0], t[1], t[2]])
    C = np.stack(rows)
    return monos, C.reshape(len(monos) * L, NB)


_MONOS, _CML = _build_monos_and_cml()
M = len(_MONOS)


def _prep_body(c_ref, w_ref, out_ref):
    out_ref[0] = jnp.dot(c_ref[...], w_ref[0],
                         preferred_element_type=jnp.float32)


def _compute_au(w):
    return pl.pallas_call(
        _prep_body,
        grid=(S,),
        in_specs=[
            pl.BlockSpec((M * L, NB), lambda s: (0, 0)),
            pl.BlockSpec((1, NB, F), lambda s: (s, 0, 0)),
        ],
        out_specs=pl.BlockSpec((1, M * L, F), lambda s: (s, 0, 0)),
        out_shape=jax.ShapeDtypeStruct((S, M * L, F), jnp.float32),
    )(jnp.asarray(_CML), w)


def _sc_body(x_hbm, cnt_hbm, au_hbm, y_hbm, x_t, y_t, a_t, cnt_t):
    wid = lax.axis_index("s") * 2 + lax.axis_index("c")
    base = wid * NPW

    pltpu.sync_copy(cnt_hbm, cnt_t)
    lane = lax.iota(jnp.int32, LANES)

    xcol = [[lane * D + (c * LANES * D + i) for i in range(D)]
            for c in range(FC)]
    ycol = [[lane * L + (c * LANES * L + l) for l in range(L)]
            for c in range(FC)]

    def tile_body(t, carry):
        b0 = base + t * TILE
        pltpu.sync_copy(x_hbm.at[pl.ds(b0, TILE)], x_t)

        def sp_body(s, start):
            cvec = plsc.load_gather(cnt_t, [jnp.full((LANES,), s, jnp.int32)])
            end = start + cvec[0]
            end = jnp.where(s == S - 1, jnp.maximum(end, N), end)
            lo = jnp.maximum(start, b0)
            hi = jnp.minimum(end, b0 + TILE)

            @pl.when(hi > lo)
            def _():
                pltpu.sync_copy(au_hbm.at[s], a_t)
                for c in range(FC):
                    for l in range(L):
                        coefs = [a_t[m * L + l, pl.ds(c * LANES, LANES)]
                                 for m in range(M)]

                        @plsc.parallel_loop(lo, hi, unroll=4)
                        def z_body(z, c=c, l=l, coefs=coefs):
                            zl = z - b0
                            row = jnp.full((LANES,), zl, jnp.int32)
                            xi = [plsc.load_gather(x_t, [row, xcol[c][i]])
                                  for i in range(D)]
                            acc = coefs[0] * xi[0]
                            for i in range(1, D):
                                acc = acc + coefs[i] * xi[i]
                            mi = D
                            t2 = D + 10
                            for (i, j) in cwr(range(D), 2):
                                p2 = xi[i] * xi[j]
                                acc = acc + coefs[mi] * p2
                                mi += 1
                                for k in range(j, D):
                                    acc = acc + coefs[t2] * (p2 * xi[k])
                                    t2 += 1
                            plsc.store_scatter(y_t, [row, ycol[c][l]], acc)
            return end

        lax.fori_loop(0, S, sp_body, 0)
        pltpu.sync_copy(y_t, y_hbm.at[pl.ds(b0, TILE)])
        return carry

    lax.fori_loop(0, NT, tile_body, 0)


_sc_main = functools.partial(
    pl.kernel,
    out_type=jax.ShapeDtypeStruct((N_SC, F * L), jnp.float32),
    mesh=plsc.VectorSubcoreMesh(core_axis_name="c", subcore_axis_name="s"),
    scratch_types=[
        pltpu.VMEM((TILE, F * D), jnp.float32),
        pltpu.VMEM((TILE, F * L), jnp.float32),
        pltpu.VMEM((M * L, F), jnp.float32),
        pltpu.VMEM((LANES,), jnp.int32),
    ],
    compiler_params=pltpu.CompilerParams(use_tc_tiling_on_sc=False,
                                         needs_layout_passes=False),
)(_sc_body)


def _tc_body(cnt_ref, x_ref, au_ref, y_ref, y4, mono, a_scr):
    row0 = N_SC + pl.program_id(0) * BN_TC
    xr = x_ref[...]

    erow = lax.broadcasted_iota(jnp.int32, (F * D, F), 0)
    ecol = lax.broadcasted_iota(jnp.int32, (F * D, F), 1)
    xi = [jnp.dot(xr, (erow == ecol * D + i).astype(jnp.float32),
                  preferred_element_type=jnp.float32) for i in range(D)]

    pairs = list(cwr(range(D), 2))
    pidx = {p: D + n for n, p in enumerate(pairs)}
    for i in range(D):
        mono[i] = xi[i]
    for (i, j) in pairs:
        mono[pidx[(i, j)]] = xi[i] * xi[j]
    for n, (i, j, k) in enumerate(cwr(range(D), 3)):
        mono[D + len(pairs) + n] = mono[pidx[(i, j)]] * xi[k]

    rowids = row0 + lax.broadcasted_iota(jnp.int32, (BN_TC, F), 0)
    y4[...] = jnp.zeros((L, BN_TC, F), jnp.float32)

    def sp_body(s, start):
        end = start + cnt_ref[s]
        end = jnp.where(s == S - 1, jnp.maximum(end, N), end)
        hit = (jnp.minimum(end, row0 + BN_TC) > jnp.maximum(start, row0))

        @pl.when(hit)
        def _():
            a_scr[...] = au_ref[s]
            mask = ((rowids >= start) & (rowids < end)).astype(jnp.float32)
            for l in range(L):
                acc = mono[0] * a_scr[l:l + 1, :]
                for m in range(1, M):
                    acc = acc + mono[m] * a_scr[m * L + l:m * L + l + 1, :]
                y4[l] = y4[l] + mask * acc

        return end

    lax.fori_loop(0, S, sp_body, jnp.int32(0))

    grow = lax.broadcasted_iota(jnp.int32, (F, F * L), 0)
    gcol = lax.broadcasted_iota(jnp.int32, (F, F * L), 1)
    out = jnp.dot(y4[0], (gcol == grow * L).astype(jnp.float32),
                  preferred_element_type=jnp.float32)
    for l in range(1, L):
        out = out + jnp.dot(y4[l], (gcol == grow * L + l).astype(jnp.float32),
                            preferred_element_type=jnp.float32)
    y_ref[...] = out


def _tc_main(xf, cnt, au):
    return pl.pallas_call(
        _tc_body,
        grid=(N_TC // BN_TC,),
        in_specs=[
            pl.BlockSpec(memory_space=pltpu.SMEM),
            pl.BlockSpec((BN_TC, F * D), lambda b: (N_SC // BN_TC + b, 0)),
            pl.BlockSpec((S, M * L, F), lambda b: (0, 0, 0)),
        ],
        out_specs=pl.BlockSpec((BN_TC, F * L), lambda b: (b, 0)),
        out_shape=jax.ShapeDtypeStruct((N_TC, F * L), jnp.float32),
        scratch_shapes=[pltpu.VMEM((L, BN_TC, F), jnp.float32),
                        pltpu.VMEM((M, BN_TC, F), jnp.float32),
                        pltpu.VMEM((M * L, F), jnp.float32)],
        compiler_params=pltpu.CompilerParams(
            vmem_limit_bytes=100 * 1024 * 1024),
    )(cnt, xf, au)


@jax.jit
def kernel(x, num_species_counts, w):
    au = _compute_au(w)
    cnt = jnp.zeros((LANES,), jnp.int32).at[:S].set(num_species_counts)
    xf = x.reshape(N, F * D)
    y_sc = _sc_main(xf[:N_SC], cnt, au)
    y_tc = _tc_main(xf, cnt, au)
    y = jnp.concatenate([y_sc, y_tc], axis=0)
    return y.reshape(N, F, L)


# speedup vs baseline: 1.0530x; 1.0530x over previous
import functools
import math
from collections import Counter
from itertools import combinations_with_replacement as cwr

import numpy as np
import jax
import jax.numpy as jnp
from jax import lax
from jax.experimental import pallas as pl
from jax.experimental.pallas import tpu as pltpu
from jax.experimental.pallas import tpu_sc as plsc

N = 8192
F = 64
D = 4
L = 4
S = 10
N1, N2, N3 = 4, 8, 12
NB = N1 + N2 + N3

N_SC = 2048
N_TC = N - N_SC
BN_TC = 256

NW = 32
NPW = N_SC // NW
TILE = 64
NT = NPW // TILE

LANES = 16
FC = F // LANES


def _build_monos_and_cml():
    rng = np.random.RandomState(42)
    P = (rng.randn(NB, NB) / np.sqrt(NB)).astype(np.float32)
    U1 = rng.randn(D, L, N1).astype(np.float32)
    U2 = rng.randn(D, D, L, N2).astype(np.float32)
    U2 = 0.5 * (U2 + U2.transpose(1, 0, 2, 3))
    U3 = rng.randn(D, D, D, L, N3).astype(np.float32)
    perms = [(0, 1, 2), (0, 2, 1), (1, 0, 2), (1, 2, 0), (2, 0, 1), (2, 1, 0)]
    U3 = sum(U3.transpose(p + (3, 4)) for p in perms) / 6.0

    K1 = np.einsum('ilb,ab->ila', U1, P[:, :N1])
    K2 = np.einsum('ijlb,ab->ijla', U2, P[:, N1:N1 + N2])
    K3 = np.einsum('ijklb,ab->ijkla', U3, P[:, N1 + N2:])

    monos = []
    rows = []
    for i in range(D):
        monos.append((i,))
        rows.append(K1[i])
    for (i, j) in cwr(range(D), 2):
        mult = 1 if i == j else 2
        monos.append((i, j))
        rows.append(mult * K2[i, j])
    for t in cwr(range(D), 3):
        c = Counter(t)
        mult = math.factorial(3)
        for v in c.values():
            mult //= math.factorial(v)
        monos.append(t)
        rows.append(mult * K3[t[0], t[1], t[2]])
    C = np.stack(rows)
    return monos, C.reshape(len(monos) * L, NB)


_MONOS, _CML = _build_monos_and_cml()
M = len(_MONOS)


def _prep_body(c_ref, w_ref, out_ref):
    out_ref[0] = jnp.dot(c_ref[...], w_ref[0],
                         preferred_element_type=jnp.float32)


def _compute_au(w):
    return pl.pallas_call(
        _prep_body,
        grid=(S,),
        in_specs=[
            pl.BlockSpec((M * L, NB), lambda s: (0, 0)),
            pl.BlockSpec((1, NB, F), lambda s: (s, 0, 0)),
        ],
        out_specs=pl.BlockSpec((1, M * L, F), lambda s: (s, 0, 0)),
        out_shape=jax.ShapeDtypeStruct((S, M * L, F), jnp.float32),
    )(jnp.asarray(_CML), w)


def _sc_body(x_hbm, cnt_hbm, au_hbm, y_hbm, x_t, y_t, a_t, cnt_t):
    wid = lax.axis_index("s") * 2 + lax.axis_index("c")
    base = wid * NPW

    pltpu.sync_copy(cnt_hbm, cnt_t)
    lane = lax.iota(jnp.int32, LANES)

    xcol = [[lane * D + (c * LANES * D + i) for i in range(D)]
            for c in range(FC)]
    ycol = [[lane * L + (c * LANES * L + l) for l in range(L)]
            for c in range(FC)]

    def tile_body(t, carry):
        b0 = base + t * TILE
        pltpu.sync_copy(x_hbm.at[pl.ds(b0, TILE)], x_t)

        def sp_body(s, start):
            cvec = plsc.load_gather(cnt_t, [jnp.full((LANES,), s, jnp.int32)])
            end = start + cvec[0]
            end = jnp.where(s == S - 1, jnp.maximum(end, N), end)
            lo = jnp.maximum(start, b0)
            hi = jnp.minimum(end, b0 + TILE)

            @pl.when(hi > lo)
            def _():
                pltpu.sync_copy(au_hbm.at[s], a_t)
                for c in range(FC):
                    for l in range(L):
                        coefs = [a_t[m * L + l, pl.ds(c * LANES, LANES)]
                                 for m in range(M)]

                        @plsc.parallel_loop(lo, hi, unroll=4)
                        def z_body(z, c=c, l=l, coefs=coefs):
                            zl = z - b0
                            row = jnp.full((LANES,), zl, jnp.int32)
                            xi = [plsc.load_gather(x_t, [row, xcol[c][i]])
                                  for i in range(D)]
                            acc = coefs[0] * xi[0]
                            for i in range(1, D):
                                acc = acc + coefs[i] * xi[i]
                            mi = D
                            t2 = D + 10
                            for (i, j) in cwr(range(D), 2):
                                p2 = xi[i] * xi[j]
                                acc = acc + coefs[mi] * p2
                                mi += 1
                                for k in range(j, D):
                                    acc = acc + coefs[t2] * (p2 * xi[k])
                                    t2 += 1
                            plsc.store_scatter(y_t, [row, ycol[c][l]], acc)
            return end

        lax.fori_loop(0, S, sp_body, 0)
        pltpu.sync_copy(y_t, y_hbm.at[pl.ds(b0, TILE)])
        return carry

    lax.fori_loop(0, NT, tile_body, 0)


_sc_main = functools.partial(
    pl.kernel,
    out_type=jax.ShapeDtypeStruct((N_SC, F * L), jnp.float32),
    mesh=plsc.VectorSubcoreMesh(core_axis_name="c", subcore_axis_name="s"),
    scratch_types=[
        pltpu.VMEM((TILE, F * D), jnp.float32),
        pltpu.VMEM((TILE, F * L), jnp.float32),
        pltpu.VMEM((M * L, F), jnp.float32),
        pltpu.VMEM((LANES,), jnp.int32),
    ],
    compiler_params=pltpu.CompilerParams(use_tc_tiling_on_sc=False,
                                         needs_layout_passes=False),
)(_sc_body)


def _tc_body(cnt_ref, x_ref, au_ref, y_ref, y4, mono, a_scr):
    row0 = N_SC + pl.program_id(0) * BN_TC
    xr = x_ref[...]

    erow = lax.broadcasted_iota(jnp.int32, (F * D, F), 0)
    ecol = lax.broadcasted_iota(jnp.int32, (F * D, F), 1)
    xi = [jnp.dot(xr, (erow == ecol * D + i).astype(jnp.float32),
                  preferred_element_type=jnp.float32) for i in range(D)]

    pairs = list(cwr(range(D), 2))
    pidx = {p: D + n for n, p in enumerate(pairs)}
    for i in range(D):
        mono[i] = xi[i]
    for (i, j) in pairs:
        mono[pidx[(i, j)]] = xi[i] * xi[j]
    for n, (i, j, k) in enumerate(cwr(range(D), 3)):
        mono[D + len(pairs) + n] = mono[pidx[(i, j)]] * xi[k]

    rowids = row0 + lax.broadcasted_iota(jnp.int32, (BN_TC, F), 0)
    y4[...] = jnp.zeros((L, BN_TC, F), jnp.float32)

    def sp_body(s, start):
        end = start + cnt_ref[s]
        end = jnp.where(s == S - 1, jnp.maximum(end, N), end)
        hit = (jnp.minimum(end, row0 + BN_TC) > jnp.maximum(start, row0))

        @pl.when(hit)
        def _():
            a_scr[...] = au_ref[s]
            mask = ((rowids >= start) & (rowids < end)).astype(jnp.float32)
            half = M // 2
            for l in range(L):
                acc_a = mono[0] * a_scr[l:l + 1, :]
                for m in range(1, half):
                    acc_a = acc_a + mono[m] * a_scr[m * L + l:m * L + l + 1, :]
                acc_b = mono[half] * a_scr[half * L + l:half * L + l + 1, :]
                for m in range(half + 1, M):
                    acc_b = acc_b + mono[m] * a_scr[m * L + l:m * L + l + 1, :]
                y4[l] = y4[l] + mask * (acc_a + acc_b)

        return end

    lax.fori_loop(0, S, sp_body, jnp.int32(0))

    grow = lax.broadcasted_iota(jnp.int32, (F, F * L), 0)
    gcol = lax.broadcasted_iota(jnp.int32, (F, F * L), 1)
    out = jnp.dot(y4[0], (gcol == grow * L).astype(jnp.float32),
                  preferred_element_type=jnp.float32)
    for l in range(1, L):
        out = out + jnp.dot(y4[l], (gcol == grow * L + l).astype(jnp.float32),
                            preferred_element_type=jnp.float32)
    y_ref[...] = out


def _tc_main(xf, cnt, au):
    return pl.pallas_call(
        _tc_body,
        grid=(N_TC // BN_TC,),
        in_specs=[
            pl.BlockSpec(memory_space=pltpu.SMEM),
            pl.BlockSpec((BN_TC, F * D), lambda b: (N_SC // BN_TC + b, 0)),
            pl.BlockSpec((S, M * L, F), lambda b: (0, 0, 0)),
        ],
        out_specs=pl.BlockSpec((BN_TC, F * L),
                               lambda b: (N_SC // BN_TC + b, 0)),
        out_shape=jax.ShapeDtypeStruct((N, F * L), jnp.float32),
        scratch_shapes=[pltpu.VMEM((L, BN_TC, F), jnp.float32),
                        pltpu.VMEM((M, BN_TC, F), jnp.float32),
                        pltpu.VMEM((M * L, F), jnp.float32)],
        compiler_params=pltpu.CompilerParams(
            vmem_limit_bytes=100 * 1024 * 1024),
    )(cnt, xf, au)


@jax.jit
def kernel(x, num_species_counts, w):
    au = _compute_au(w)
    cnt = jnp.zeros((LANES,), jnp.int32).at[:S].set(num_species_counts)
    xf = x.reshape(N, F * D)
    y_sc = _sc_main(xf, cnt, au)
    y_tc = _tc_main(xf, cnt, au)
    y = lax.dynamic_update_slice(y_tc, y_sc, (0, 0))
    return y.reshape(N, F, L)
